# baseline (speedup 1.0000x reference)
import functools

import jax
import jax.numpy as jnp
from jax import lax
from jax.experimental import pallas as pl
from jax.experimental.pallas import tpu as pltpu
from jax.experimental.pallas import tpu_sc as plsc

N = 10000
NP = 10240
E = 320000
G = 64
G1 = 72
CH = 128
NCH = 80
NCH2 = NCH + 4
EPT = NCH * CH
EP = 32 * EPT
NPT = NP // 32
RPT = NP // 16
BR = 1024

_mesh = plsc.VectorSubcoreMesh(core_axis_name="c", subcore_axis_name="s")
_sc_params = pltpu.CompilerParams(use_tc_tiling_on_sc=False)


@functools.partial(
    pl.kernel,
    mesh=_mesh,
    compiler_params=_sc_params,
    out_type=jax.ShapeDtypeStruct((2, NP, 16), jnp.float32),
    scratch_types=[
        pltpu.VMEM((NCH2, CH), jnp.int32),
        pltpu.VMEM((CH, 16), jnp.float32),
        pltpu.VMEM_SHARED((NP, 16), jnp.float32),
    ],
)
def _sc_degree(cols_hbm, ones_hbm, zeros_hbm, out_hbm, cols_v, ones_v, acc):
    c = lax.axis_index("c")
    s = lax.axis_index("s")
    wid = c * 16 + s
    pltpu.sync_copy(zeros_hbm.at[pl.ds(s * RPT, RPT)], acc.at[pl.ds(s * RPT, RPT)])
    pltpu.sync_copy(cols_hbm.at[wid], cols_v)
    pltpu.sync_copy(ones_hbm, ones_v)
    plsc.subcore_barrier()

    def body(j, carry):
        pltpu.sync_copy(ones_v, acc.at[cols_v.at[j]], add=True)
        return carry

    lax.fori_loop(0, NCH, body, 0)
    plsc.subcore_barrier()
    pltpu.sync_copy(acc.at[pl.ds(s * RPT, RPT)], out_hbm.at[c, pl.ds(s * RPT, RPT)])


def _chunk(D):
    return 64 if D == 128 else 128


def _nbuf(D):
    return 2 if D == 128 else 4


def _make_sc_scatter(D):
    CHD = _chunk(D)
    NB = _nbuf(D)
    NCHD = EPT // CHD
    NCHD2 = NCHD + 4

    @functools.partial(
        pl.kernel,
        mesh=_mesh,
        compiler_params=_sc_params,
        out_type=jax.ShapeDtypeStruct((2, NP, D), jnp.float32),
        scratch_types=[
            pltpu.VMEM((NCHD2, CHD), jnp.int32),
            pltpu.VMEM((NCHD2, CHD), jnp.int32),
        ] + [pltpu.VMEM((CHD, D), jnp.float32) for _ in range(NB)] + [
            pltpu.VMEM_SHARED((NP, D), jnp.float32),
        ] + [pltpu.SemaphoreType.DMA for _ in range(2 * NB)],
    )
    def scat(g_hbm, rows_hbm, cols_hbm, zeros_hbm, out_hbm,
             rows_v, cols_v, *rest):
        bufs = rest[:NB]
        acc = rest[NB]
        gsem = rest[NB + 1:2 * NB + 1]
        ssem = rest[2 * NB + 1:]
        c = lax.axis_index("c")
        s = lax.axis_index("s")
        wid = c * 16 + s
        pltpu.sync_copy(zeros_hbm.at[pl.ds(s * RPT, RPT)], acc.at[pl.ds(s * RPT, RPT)])
        pltpu.sync_copy(rows_hbm.at[wid], rows_v)
        pltpu.sync_copy(cols_hbm.at[wid], cols_v)
        plsc.subcore_barrier()

        for b in range(NB):
            pltpu.async_copy(g_hbm.at[rows_v.at[b]], bufs[b], gsem[b])

        def body(jj, carry):
            j = NB * jj
            for b in range(NB):
                pltpu.make_async_copy(g_hbm.at[rows_v.at[j + b]], bufs[b],
                                      gsem[b]).wait()
                pltpu.async_copy(bufs[b], acc.at[cols_v.at[j + b]], ssem[b],
                                 add=True)
            for b in range(NB):
                pltpu.make_async_copy(g_hbm.at[rows_v.at[j + b]], bufs[b],
                                      ssem[b]).wait()
                pltpu.async_copy(g_hbm.at[rows_v.at[j + b + NB]], bufs[b],
                                 gsem[b])
            return carry

        lax.fori_loop(0, NCHD // NB, body, 0)
        for b in range(NB):
            pltpu.make_async_copy(g_hbm.at[rows_v.at[NCHD + b]], bufs[b],
                                  gsem[b]).wait()
        plsc.subcore_barrier()
        pltpu.sync_copy(acc.at[pl.ds(s * RPT, RPT)], out_hbm.at[c, pl.ds(s * RPT, RPT)])

    return scat


_sc_scatter = {D: _make_sc_scatter(D) for D in (128, 64, 32, 16)}


@functools.partial(
    pl.kernel,
    mesh=_mesh,
    out_type=[
        jax.ShapeDtypeStruct((32, G1 * 16), jnp.float32),
        jax.ShapeDtypeStruct((32, G1 * 16), jnp.float32),
        jax.ShapeDtypeStruct((32, G1 * 16), jnp.float32),
    ],
    scratch_types=[
        pltpu.VMEM((NPT * 16,), jnp.float32),
        pltpu.VMEM((NPT,), jnp.int32),
        pltpu.VMEM((G1 * 16,), jnp.float32),
        pltpu.VMEM((G1 * 16,), jnp.float32),
        pltpu.VMEM((G1 * 16,), jnp.float32),
    ],
)
def _sc_pool(h_hbm, bidx_hbm, maxo, sumo, cnto, h_v, b_v, macc, sacc, cacc):
    c = lax.axis_index("c")
    s = lax.axis_index("s")
    wid = c * 16 + s
    pltpu.sync_copy(h_hbm.at[pl.ds(wid * NPT * 16, NPT * 16)], h_v)
    pltpu.sync_copy(bidx_hbm.at[pl.ds(wid * NPT, NPT)], b_v)
    neg = jnp.full((16,), -jnp.inf, jnp.float32)
    zero = jnp.zeros((16,), jnp.float32)
    one = jnp.ones((16,), jnp.float32)

    def init(j, carry):
        macc[pl.ds(j * 16, 16)] = neg
        sacc[pl.ds(j * 16, 16)] = zero
        cacc[pl.ds(j * 16, 16)] = zero
        return carry

    lax.fori_loop(0, G1, init, 0)

    def body(i16, carry):
        bs = b_v[pl.ds(i16 * 16, 16)]
        for l in range(16):
            off = bs[l] * 16
            v = h_v[pl.ds((i16 * 16 + l) * 16, 16)]
            macc[pl.ds(off, 16)] = jnp.maximum(macc[pl.ds(off, 16)], v)
            sacc[pl.ds(off, 16)] = sacc[pl.ds(off, 16)] + v
            cacc[pl.ds(off, 16)] = cacc[pl.ds(off, 16)] + one
        return carry

    lax.fori_loop(0, NPT // 16, body, 0)
    pltpu.sync_copy(macc, maxo.at[wid])
    pltpu.sync_copy(sacc, sumo.at[wid])
    pltpu.sync_copy(cacc, cnto.at[wid])


def _tc_first(x_p, wt0, deg2):
    def body(x_ref, w_ref, d_ref, g_ref, dinv_ref):
        pid = pl.program_id(0)
        deg = d_ref[0, :, 0:1] + d_ref[1, :, 0:1] + 1.0
        rid = pid * BR + lax.broadcasted_iota(jnp.int32, (BR, 1), 0)
        dinv = jnp.where(rid < N, lax.rsqrt(deg), 0.0)
        g_ref[...] = dinv * jnp.dot(x_ref[...], w_ref[...],
                                    preferred_element_type=jnp.float32)
        dinv_ref[...] = dinv

    return pl.pallas_call(
        body,
        grid=(NP // BR,),
        in_specs=[
            pl.BlockSpec((BR, 128), lambda i: (i, 0)),
            pl.BlockSpec((128, 128), lambda i: (0, 0)),
            pl.BlockSpec((2, BR, 16), lambda i: (0, i, 0)),
        ],
        out_specs=[
            pl.BlockSpec((BR, 128), lambda i: (i, 0)),
            pl.BlockSpec((BR, 1), lambda i: (i, 0)),
        ],
        out_shape=[
            jax.ShapeDtypeStruct((NP, 128), jnp.float32),
            jax.ShapeDtypeStruct((NP, 1), jnp.float32),
        ],
    )(x_p, wt0, deg2)


def _tc_mid(s2, gprev, dinv, bvec, wt, Dp, Dn):
    def body(s_ref, g_ref, d_ref, b_ref, w_ref, o_ref):
        dv = d_ref[...]
        t = jnp.tanh(dv * (s_ref[0] + s_ref[1] + g_ref[...]) + b_ref[...])
        o_ref[...] = dv * jnp.dot(t, w_ref[...], preferred_element_type=jnp.float32)

    return pl.pallas_call(
        body,
        grid=(NP // BR,),
        in_specs=[
            pl.BlockSpec((2, BR, Dp), lambda i: (0, i, 0)),
            pl.BlockSpec((BR, Dp), lambda i: (i, 0)),
            pl.BlockSpec((BR, 1), lambda i: (i, 0)),
            pl.BlockSpec((1, Dp), lambda i: (0, 0)),
            pl.BlockSpec((Dp, Dn), lambda i: (0, 0)),
        ],
        out_specs=pl.BlockSpec((BR, Dn), lambda i: (i, 0)),
        out_shape=jax.ShapeDtypeStruct((NP, Dn), jnp.float32),
    )(s2, gprev, dinv, bvec, wt)


def _tc_fin(s2, gprev, dinv, bvec):
    def body(s_ref, g_ref, d_ref, b_ref, o_ref):
        dv = d_ref[...]
        o_ref[...] = jnp.tanh(dv * (s_ref[0] + s_ref[1] + g_ref[...]) + b_ref[...])

    return pl.pallas_call(
        body,
        grid=(NP // BR,),
        in_specs=[
            pl.BlockSpec((2, BR, 16), lambda i: (0, i, 0)),
            pl.BlockSpec((BR, 16), lambda i: (i, 0)),
            pl.BlockSpec((BR, 1), lambda i: (i, 0)),
            pl.BlockSpec((1, 16), lambda i: (0, 0)),
        ],
        out_specs=pl.BlockSpec((BR, 16), lambda i: (i, 0)),
        out_shape=jax.ShapeDtypeStruct((NP, 16), jnp.float32),
    )(s2, gprev, dinv, bvec)


def _tc_final(maxp, sump, cntp, wmax, wmean, bout2):
    def body(m_ref, s_ref, c_ref, wx_ref, wm_ref, b_ref, o_ref):
        mx = jnp.max(m_ref[...], axis=0)[:G, :]
        sm = jnp.sum(s_ref[...], axis=0)[:G, :]
        ct = jnp.sum(c_ref[...], axis=0)[:G, :]
        mean = sm / jnp.maximum(ct, 1.0)
        o_ref[...] = (jnp.dot(mx, wx_ref[...], preferred_element_type=jnp.float32)
                      + jnp.dot(mean, wm_ref[...], preferred_element_type=jnp.float32)
                      + b_ref[...])

    full3 = pl.BlockSpec((32, G1, 16), lambda: (0, 0, 0))
    return pl.pallas_call(
        body,
        in_specs=[
            full3, full3, full3,
            pl.BlockSpec((16, 1), lambda: (0, 0)),
            pl.BlockSpec((16, 1), lambda: (0, 0)),
            pl.BlockSpec((1, 1), lambda: (0, 0)),
        ],
        out_specs=pl.BlockSpec((G, 1), lambda: (0, 0)),
        out_shape=jax.ShapeDtypeStruct((G, 1), jnp.float32),
    )(maxp, sump, cntp, wmax, wmean, bout2)


def kernel(x, edge_index, batch_index, W0, b0, W1, b1, W2, b2, W3, b3, Wout, bout):
    f32 = jnp.float32
    x_p = jnp.pad(x, ((0, NP - N), (0, 0)))
    rows = edge_index[0]
    cols = edge_index[1]
    pad_ids = (N + (jnp.arange(EP - E, dtype=jnp.int32) % (NP - N))).astype(jnp.int32)
    rows_f = jnp.concatenate([rows, pad_ids])
    cols_f = jnp.concatenate([cols, pad_ids])

    def chunked(flat, chd):
        nchd = EPT // chd
        dummy = (N + (jnp.arange(32 * 4 * chd, dtype=jnp.int32) % (NP - N))
                 ).reshape(32, 4, chd).astype(jnp.int32)
        return jnp.concatenate([flat.reshape(32, nchd, chd), dummy], axis=1)

    rows_by_ch = {chd: chunked(rows_f, chd) for chd in (64, 128)}
    cols_by_ch = {chd: chunked(cols_f, chd) for chd in (64, 128)}
    bidx_p = jnp.concatenate(
        [batch_index.astype(jnp.int32), jnp.full((NP - N,), G, jnp.int32)])
    ones16 = jnp.ones((CH, 16), f32)
    zeros16 = jnp.zeros((NP, 16), f32)

    deg2 = _sc_degree(cols_by_ch[128], ones16, zeros16)
    g0, dinv = _tc_first(x_p, W0.T, deg2)
    s0 = _sc_scatter[128](g0, rows_by_ch[64], cols_by_ch[64],
                          jnp.zeros((NP, 128), f32))
    g1 = _tc_mid(s0, g0, dinv, b0.reshape(1, -1), W1.T, 128, 64)
    s1 = _sc_scatter[64](g1, rows_by_ch[128], cols_by_ch[128],
                         jnp.zeros((NP, 64), f32))
    g2 = _tc_mid(s1, g1, dinv, b1.reshape(1, -1), W2.T, 64, 32)
    s2 = _sc_scatter[32](g2, rows_by_ch[128], cols_by_ch[128],
                         jnp.zeros((NP, 32), f32))
    g3 = _tc_mid(s2, g2, dinv, b2.reshape(1, -1), W3.T, 32, 16)
    s3 = _sc_scatter[16](g3, rows_by_ch[128], cols_by_ch[128], zeros16)
    h4 = _tc_fin(s3, g3, dinv, b3.reshape(1, -1))

    mx, sm, ct = _sc_pool(h4.reshape(-1), bidx_p)
    out = _tc_final(mx.reshape(32, G1, 16), sm.reshape(32, G1, 16),
                    ct.reshape(32, G1, 16),
                    Wout[:, :16].T, Wout[:, 16:].T, bout.reshape(1, 1))
    return out


# speedup vs baseline: 1.0189x; 1.0189x over previous
import functools

import jax
import jax.numpy as jnp
from jax import lax
from jax.experimental import pallas as pl
from jax.experimental.pallas import tpu as pltpu
from jax.experimental.pallas import tpu_sc as plsc

N = 10000
NP = 10240
E = 320000
G = 64
G1 = 72
CH = 128
NCH = 80
NCH2 = NCH + 4
EPT = NCH * CH
EP = 32 * EPT
NPT = NP // 32
RPT = NP // 16
BR = 1024

_mesh = plsc.VectorSubcoreMesh(core_axis_name="c", subcore_axis_name="s")
_sc_params = pltpu.CompilerParams(use_tc_tiling_on_sc=False)


@functools.partial(
    pl.kernel,
    mesh=_mesh,
    compiler_params=_sc_params,
    out_type=jax.ShapeDtypeStruct((2, NP, 16), jnp.float32),
    scratch_types=[
        pltpu.VMEM((NCH2, CH), jnp.int32),
        pltpu.VMEM((CH, 16), jnp.float32),
        pltpu.VMEM_SHARED((NP, 16), jnp.float32),
    ],
)
def _sc_degree(cols_hbm, ones_hbm, zeros_hbm, out_hbm, cols_v, ones_v, acc):
    c = lax.axis_index("c")
    s = lax.axis_index("s")
    wid = c * 16 + s
    pltpu.sync_copy(zeros_hbm.at[pl.ds(s * RPT, RPT)], acc.at[pl.ds(s * RPT, RPT)])
    pltpu.sync_copy(cols_hbm.at[wid], cols_v)
    pltpu.sync_copy(ones_hbm, ones_v)
    plsc.subcore_barrier()

    def body(j, carry):
        pltpu.sync_copy(ones_v, acc.at[cols_v.at[j]], add=True)
        return carry

    lax.fori_loop(0, NCH, body, 0)
    plsc.subcore_barrier()
    pltpu.sync_copy(acc.at[pl.ds(s * RPT, RPT)], out_hbm.at[c, pl.ds(s * RPT, RPT)])


def _chunk(D):
    return 64 if D == 128 else 128


def _nbuf(D):
    return 2 if D == 128 else 4


def _make_sc_scatter(D):
    CHD = _chunk(D)
    NB = _nbuf(D)
    NCHD = EPT // CHD
    NCHD2 = NCHD + 4

    @functools.partial(
        pl.kernel,
        mesh=_mesh,
        compiler_params=_sc_params,
        out_type=jax.ShapeDtypeStruct((2, NP, D), jnp.float32),
        scratch_types=[
            pltpu.VMEM((NCHD2, CHD), jnp.int32),
            pltpu.VMEM((NCHD2, CHD), jnp.int32),
        ] + [pltpu.VMEM((CHD, D), jnp.float32) for _ in range(NB)] + [
            pltpu.VMEM_SHARED((NP, D), jnp.float32),
        ] + [pltpu.SemaphoreType.DMA for _ in range(2 * NB)],
    )
    def scat(g_hbm, rows_hbm, cols_hbm, zeros_hbm, out_hbm,
             rows_v, cols_v, *rest):
        bufs = rest[:NB]
        acc = rest[NB]
        gsem = rest[NB + 1:2 * NB + 1]
        ssem = rest[2 * NB + 1:]
        c = lax.axis_index("c")
        s = lax.axis_index("s")
        wid = c * 16 + s
        pltpu.sync_copy(zeros_hbm.at[pl.ds(s * RPT, RPT)], acc.at[pl.ds(s * RPT, RPT)])
        pltpu.sync_copy(rows_hbm.at[wid], rows_v)
        pltpu.sync_copy(cols_hbm.at[wid], cols_v)
        plsc.subcore_barrier()

        for b in range(NB):
            pltpu.async_copy(g_hbm.at[rows_v.at[b]], bufs[b], gsem[b])

        def body(jj, carry):
            j = NB * jj
            for b in range(NB):
                pltpu.make_async_copy(g_hbm.at[rows_v.at[j + b]], bufs[b],
                                      gsem[b]).wait()
                pltpu.async_copy(bufs[b], acc.at[cols_v.at[j + b]], ssem[b],
                                 add=True)
            for b in range(NB):
                pltpu.make_async_copy(g_hbm.at[rows_v.at[j + b]], bufs[b],
                                      ssem[b]).wait()
                pltpu.async_copy(g_hbm.at[rows_v.at[j + b + NB]], bufs[b],
                                 gsem[b])
            return carry

        lax.fori_loop(0, NCHD // NB, body, 0)
        for b in range(NB):
            pltpu.make_async_copy(g_hbm.at[rows_v.at[NCHD + b]], bufs[b],
                                  gsem[b]).wait()
        plsc.subcore_barrier()
        pltpu.sync_copy(acc.at[pl.ds(s * RPT, RPT)], out_hbm.at[c, pl.ds(s * RPT, RPT)])

    return scat


_sc_scatter = {D: _make_sc_scatter(D) for D in (128, 64, 32, 16)}


@functools.partial(
    pl.kernel,
    mesh=_mesh,
    out_type=[
        jax.ShapeDtypeStruct((32, G1 * 16), jnp.float32),
        jax.ShapeDtypeStruct((32, G1 * 16), jnp.float32),
        jax.ShapeDtypeStruct((32, G1 * 16), jnp.float32),
    ],
    scratch_types=[
        pltpu.VMEM((NPT * 16,), jnp.float32),
        pltpu.VMEM((NPT * 16,), jnp.float32),
        pltpu.VMEM((NPT * 16,), jnp.float32),
        pltpu.VMEM((NPT,), jnp.float32),
        pltpu.VMEM((16,), jnp.float32),
        pltpu.VMEM((NPT,), jnp.int32),
        pltpu.VMEM((G1 * 16,), jnp.float32),
        pltpu.VMEM((G1 * 16,), jnp.float32),
        pltpu.VMEM((G1 * 16,), jnp.float32),
    ],
)
def _sc_pool(s3_hbm, g3_hbm, dinv_hbm, b3_hbm, bidx_hbm, maxo, sumo, cnto,
             s0_v, s1_v, g_v, dv_v, b3_v, b_v, macc, sacc, cacc):
    c = lax.axis_index("c")
    s = lax.axis_index("s")
    wid = c * 16 + s
    base = wid * NPT * 16
    pltpu.sync_copy(s3_hbm.at[0, pl.ds(base, NPT * 16)], s0_v)
    pltpu.sync_copy(s3_hbm.at[1, pl.ds(base, NPT * 16)], s1_v)
    pltpu.sync_copy(g3_hbm.at[pl.ds(base, NPT * 16)], g_v)
    pltpu.sync_copy(dinv_hbm.at[pl.ds(wid * NPT, NPT)], dv_v)
    pltpu.sync_copy(b3_hbm, b3_v)
    pltpu.sync_copy(bidx_hbm.at[pl.ds(wid * NPT, NPT)], b_v)
    neg = jnp.full((16,), -jnp.inf, jnp.float32)
    zero = jnp.zeros((16,), jnp.float32)
    one = jnp.ones((16,), jnp.float32)

    def init(j, carry):
        macc[pl.ds(j * 16, 16)] = neg
        sacc[pl.ds(j * 16, 16)] = zero
        cacc[pl.ds(j * 16, 16)] = zero
        return carry

    lax.fori_loop(0, G1, init, 0)
    bvec = b3_v[...]

    def body(i16, carry):
        bs = b_v[pl.ds(i16 * 16, 16)]
        dvs = dv_v[pl.ds(i16 * 16, 16)]
        for l in range(16):
            off = bs[l] * 16
            r = pl.ds((i16 * 16 + l) * 16, 16)
            z = dvs[l] * (s0_v[r] + s1_v[r] + g_v[r]) + bvec
            e = jnp.exp(2.0 * z)
            v = 1.0 - 2.0 / (e + 1.0)
            macc[pl.ds(off, 16)] = jnp.maximum(macc[pl.ds(off, 16)], v)
            sacc[pl.ds(off, 16)] = sacc[pl.ds(off, 16)] + v
            cacc[pl.ds(off, 16)] = cacc[pl.ds(off, 16)] + one
        return carry

    lax.fori_loop(0, NPT // 16, body, 0)
    pltpu.sync_copy(macc, maxo.at[wid])
    pltpu.sync_copy(sacc, sumo.at[wid])
    pltpu.sync_copy(cacc, cnto.at[wid])


def _tc_first(x_p, wt0, deg2):
    def body(x_ref, w_ref, d_ref, g_ref, dinv_ref):
        pid = pl.program_id(0)
        deg = d_ref[0, :, 0:1] + d_ref[1, :, 0:1] + 1.0
        rid = pid * BR + lax.broadcasted_iota(jnp.int32, (BR, 1), 0)
        dinv = jnp.where(rid < N, lax.rsqrt(deg), 0.0)
        g_ref[...] = dinv * jnp.dot(x_ref[...], w_ref[...],
                                    preferred_element_type=jnp.float32)
        dinv_ref[...] = dinv

    return pl.pallas_call(
        body,
        grid=(NP // BR,),
        in_specs=[
            pl.BlockSpec((BR, 128), lambda i: (i, 0)),
            pl.BlockSpec((128, 128), lambda i: (0, 0)),
            pl.BlockSpec((2, BR, 16), lambda i: (0, i, 0)),
        ],
        out_specs=[
            pl.BlockSpec((BR, 128), lambda i: (i, 0)),
            pl.BlockSpec((BR, 1), lambda i: (i, 0)),
        ],
        out_shape=[
            jax.ShapeDtypeStruct((NP, 128), jnp.float32),
            jax.ShapeDtypeStruct((NP, 1), jnp.float32),
        ],
    )(x_p, wt0, deg2)


def _tc_mid(s2, gprev, dinv, bvec, wt, Dp, Dn):
    def body(s_ref, g_ref, d_ref, b_ref, w_ref, o_ref):
        dv = d_ref[...]
        t = jnp.tanh(dv * (s_ref[0] + s_ref[1] + g_ref[...]) + b_ref[...])
        o_ref[...] = dv * jnp.dot(t, w_ref[...], preferred_element_type=jnp.float32)

    return pl.pallas_call(
        body,
        grid=(NP // BR,),
        in_specs=[
            pl.BlockSpec((2, BR, Dp), lambda i: (0, i, 0)),
            pl.BlockSpec((BR, Dp), lambda i: (i, 0)),
            pl.BlockSpec((BR, 1), lambda i: (i, 0)),
            pl.BlockSpec((1, Dp), lambda i: (0, 0)),
            pl.BlockSpec((Dp, Dn), lambda i: (0, 0)),
        ],
        out_specs=pl.BlockSpec((BR, Dn), lambda i: (i, 0)),
        out_shape=jax.ShapeDtypeStruct((NP, Dn), jnp.float32),
    )(s2, gprev, dinv, bvec, wt)


def _tc_final(maxp, sump, cntp, wmax, wmean, bout2):
    def body(m_ref, s_ref, c_ref, wx_ref, wm_ref, b_ref, o_ref):
        mx = jnp.max(m_ref[...], axis=0)[:G, :]
        sm = jnp.sum(s_ref[...], axis=0)[:G, :]
        ct = jnp.sum(c_ref[...], axis=0)[:G, :]
        mean = sm / jnp.maximum(ct, 1.0)
        o_ref[...] = (jnp.dot(mx, wx_ref[...], preferred_element_type=jnp.float32)
                      + jnp.dot(mean, wm_ref[...], preferred_element_type=jnp.float32)
                      + b_ref[...])

    full3 = pl.BlockSpec((32, G1, 16), lambda: (0, 0, 0))
    return pl.pallas_call(
        body,
        in_specs=[
            full3, full3, full3,
            pl.BlockSpec((16, 1), lambda: (0, 0)),
            pl.BlockSpec((16, 1), lambda: (0, 0)),
            pl.BlockSpec((1, 1), lambda: (0, 0)),
        ],
        out_specs=pl.BlockSpec((G, 1), lambda: (0, 0)),
        out_shape=jax.ShapeDtypeStruct((G, 1), jnp.float32),
    )(maxp, sump, cntp, wmax, wmean, bout2)


def kernel(x, edge_index, batch_index, W0, b0, W1, b1, W2, b2, W3, b3, Wout, bout):
    f32 = jnp.float32
    x_p = jnp.pad(x, ((0, NP - N), (0, 0)))
    rows = edge_index[0]
    cols = edge_index[1]
    pad_ids = (N + (jnp.arange(EP - E, dtype=jnp.int32) % (NP - N))).astype(jnp.int32)
    rows_f = jnp.concatenate([rows, pad_ids])
    cols_f = jnp.concatenate([cols, pad_ids])

    def chunked(flat, chd):
        nchd = EPT // chd
        dummy = (N + (jnp.arange(32 * 4 * chd, dtype=jnp.int32) % (NP - N))
                 ).reshape(32, 4, chd).astype(jnp.int32)
        return jnp.concatenate([flat.reshape(32, nchd, chd), dummy], axis=1)

    rows_by_ch = {chd: chunked(rows_f, chd) for chd in (64, 128)}
    cols_by_ch = {chd: chunked(cols_f, chd) for chd in (64, 128)}
    bidx_p = jnp.concatenate(
        [batch_index.astype(jnp.int32), jnp.full((NP - N,), G, jnp.int32)])
    ones16 = jnp.ones((CH, 16), f32)
    zeros16 = jnp.zeros((NP, 16), f32)

    deg2 = _sc_degree(cols_by_ch[128], ones16, zeros16)
    g0, dinv = _tc_first(x_p, W0.T, deg2)
    s0 = _sc_scatter[128](g0, rows_by_ch[64], cols_by_ch[64],
                          jnp.zeros((NP, 128), f32))
    g1 = _tc_mid(s0, g0, dinv, b0.reshape(1, -1), W1.T, 128, 64)
    s1 = _sc_scatter[64](g1, rows_by_ch[128], cols_by_ch[128],
                         jnp.zeros((NP, 64), f32))
    g2 = _tc_mid(s1, g1, dinv, b1.reshape(1, -1), W2.T, 64, 32)
    s2 = _sc_scatter[32](g2, rows_by_ch[128], cols_by_ch[128],
                         jnp.zeros((NP, 32), f32))
    g3 = _tc_mid(s2, g2, dinv, b2.reshape(1, -1), W3.T, 32, 16)
    s3 = _sc_scatter[16](g3, rows_by_ch[128], cols_by_ch[128], zeros16)

    mx, sm, ct = _sc_pool(s3.reshape(2, NP * 16), g3.reshape(-1),
                          dinv.reshape(-1), b3.astype(f32), bidx_p)
    out = _tc_final(mx.reshape(32, G1, 16), sm.reshape(32, G1, 16),
                    ct.reshape(32, G1, 16),
                    Wout[:, :16].T, Wout[:, 16:].T, bout.reshape(1, 1))
    return out


# speedup vs baseline: 1.1150x; 1.0944x over previous
import functools

import jax
import jax.numpy as jnp
from jax import lax
from jax.experimental import pallas as pl
from jax.experimental.pallas import tpu as pltpu
from jax.experimental.pallas import tpu_sc as plsc

N = 10000
NP = 10240
E = 320000
G = 64
G1 = 72
CH = 128
NCH = 80
NCH2 = NCH + 4
EPT = NCH * CH
EP = 32 * EPT
NPT = NP // 32
RPT = NP // 16
BR = 1024

_mesh = plsc.VectorSubcoreMesh(core_axis_name="c", subcore_axis_name="s")
_sc_params = pltpu.CompilerParams(use_tc_tiling_on_sc=False)


@functools.partial(
    pl.kernel,
    mesh=_mesh,
    compiler_params=_sc_params,
    out_type=jax.ShapeDtypeStruct((2, NP, 16), jnp.float32),
    scratch_types=[
        pltpu.VMEM((NCH2, CH), jnp.int32),
        pltpu.VMEM((CH, 16), jnp.float32),
        pltpu.VMEM_SHARED((NP, 16), jnp.float32),
    ],
)
def _sc_degree(cols_hbm, ones_hbm, zeros_hbm, out_hbm, cols_v, ones_v, acc):
    c = lax.axis_index("c")
    s = lax.axis_index("s")
    wid = c * 16 + s
    pltpu.sync_copy(zeros_hbm.at[pl.ds(s * RPT, RPT)], acc.at[pl.ds(s * RPT, RPT)])
    pltpu.sync_copy(cols_hbm.at[wid], cols_v)
    pltpu.sync_copy(ones_hbm, ones_v)
    plsc.subcore_barrier()

    def body(j, carry):
        pltpu.sync_copy(ones_v, acc.at[cols_v.at[j]], add=True)
        return carry

    lax.fori_loop(0, NCH, body, 0)
    plsc.subcore_barrier()
    pltpu.sync_copy(acc.at[pl.ds(s * RPT, RPT)], out_hbm.at[c, pl.ds(s * RPT, RPT)])


def _chunk(D):
    return 128


def _nbuf(D):
    return 4


def _make_sc_scatter(D):
    CHD = _chunk(D)
    NB = _nbuf(D)
    NCHD = EPT // CHD
    NCHD2 = NCHD + 4

    @functools.partial(
        pl.kernel,
        mesh=_mesh,
        compiler_params=_sc_params,
        out_type=jax.ShapeDtypeStruct((2, NP, D), jnp.float32),
        scratch_types=[
            pltpu.VMEM((NCHD2, CHD), jnp.int32),
            pltpu.VMEM((NCHD2, CHD), jnp.int32),
        ] + [pltpu.VMEM((CHD, D), jnp.float32) for _ in range(NB)] + [
            pltpu.VMEM_SHARED((NP, D), jnp.float32),
        ] + [pltpu.SemaphoreType.DMA for _ in range(2 * NB)],
    )
    def scat(g_hbm, rows_hbm, cols_hbm, zeros_hbm, out_hbm,
             rows_v, cols_v, *rest):
        bufs = rest[:NB]
        acc = rest[NB]
        gsem = rest[NB + 1:2 * NB + 1]
        ssem = rest[2 * NB + 1:]
        c = lax.axis_index("c")
        s = lax.axis_index("s")
        wid = c * 16 + s
        pltpu.sync_copy(zeros_hbm.at[pl.ds(s * RPT, RPT)], acc.at[pl.ds(s * RPT, RPT)])
        pltpu.sync_copy(rows_hbm.at[wid], rows_v)
        pltpu.sync_copy(cols_hbm.at[wid], cols_v)
        plsc.subcore_barrier()

        for b in range(NB):
            pltpu.async_copy(g_hbm.at[rows_v.at[b]], bufs[b], gsem[b])

        def body(jj, carry):
            j = NB * jj
            for b in range(NB):
                pltpu.make_async_copy(g_hbm.at[rows_v.at[j + b]], bufs[b],
                                      gsem[b]).wait()
                pltpu.async_copy(bufs[b], acc.at[cols_v.at[j + b]], ssem[b],
                                 add=True)
            for b in range(NB):
                pltpu.make_async_copy(g_hbm.at[rows_v.at[j + b]], bufs[b],
                                      ssem[b]).wait()
                pltpu.async_copy(g_hbm.at[rows_v.at[j + b + NB]], bufs[b],
                                 gsem[b])
            return carry

        lax.fori_loop(0, NCHD // NB, body, 0)
        for b in range(NB):
            pltpu.make_async_copy(g_hbm.at[rows_v.at[NCHD + b]], bufs[b],
                                  gsem[b]).wait()
        plsc.subcore_barrier()
        pltpu.sync_copy(acc.at[pl.ds(s * RPT, RPT)], out_hbm.at[c, pl.ds(s * RPT, RPT)])

    return scat


_sc_scatter = {D: _make_sc_scatter(D) for D in (64, 32, 16)}

EPT0 = EP // 16
NCH0 = EPT0 // CH
NCH02 = NCH0 + 4
_NB0 = 4


@functools.partial(
    pl.kernel,
    mesh=_mesh,
    compiler_params=_sc_params,
    out_type=jax.ShapeDtypeStruct((2, NP, 64), jnp.float32),
    scratch_types=[
        pltpu.VMEM((NCH02, CH), jnp.int32),
        pltpu.VMEM((NCH02, CH), jnp.int32),
    ] + [pltpu.VMEM((CH, 64), jnp.float32) for _ in range(_NB0)] + [
        pltpu.VMEM_SHARED((NP, 64), jnp.float32),
    ] + [pltpu.SemaphoreType.DMA for _ in range(2 * _NB0)],
)
def _sc_scatter0(g_hbm, rows_hbm, cols_hbm, zeros_hbm, out_hbm,
                 rows_v, cols_v, *rest):
    bufs = rest[:_NB0]
    acc = rest[_NB0]
    gsem = rest[_NB0 + 1:2 * _NB0 + 1]
    ssem = rest[2 * _NB0 + 1:]
    c = lax.axis_index("c")
    s = lax.axis_index("s")
    tab = g_hbm.at[c]
    pltpu.sync_copy(zeros_hbm.at[pl.ds(s * RPT, RPT)], acc.at[pl.ds(s * RPT, RPT)])
    pltpu.sync_copy(rows_hbm.at[s], rows_v)
    pltpu.sync_copy(cols_hbm.at[s], cols_v)
    plsc.subcore_barrier()

    for b in range(_NB0):
        pltpu.async_copy(tab.at[rows_v.at[b]], bufs[b], gsem[b])

    def body(jj, carry):
        j = _NB0 * jj
        for b in range(_NB0):
            pltpu.make_async_copy(tab.at[rows_v.at[j + b]], bufs[b],
                                  gsem[b]).wait()
            pltpu.async_copy(bufs[b], acc.at[cols_v.at[j + b]], ssem[b],
                             add=True)
        for b in range(_NB0):
            pltpu.make_async_copy(tab.at[rows_v.at[j + b]], bufs[b],
                                  ssem[b]).wait()
            pltpu.async_copy(tab.at[rows_v.at[j + b + _NB0]], bufs[b],
                             gsem[b])
        return carry

    lax.fori_loop(0, NCH0 // _NB0, body, 0)
    for b in range(_NB0):
        pltpu.make_async_copy(tab.at[rows_v.at[NCH0 + b]], bufs[b],
                              gsem[b]).wait()
    plsc.subcore_barrier()
    pltpu.sync_copy(acc.at[pl.ds(s * RPT, RPT)], out_hbm.at[c, pl.ds(s * RPT, RPT)])


@functools.partial(
    pl.kernel,
    mesh=_mesh,
    out_type=[
        jax.ShapeDtypeStruct((32, G1 * 16), jnp.float32),
        jax.ShapeDtypeStruct((32, G1 * 16), jnp.float32),
        jax.ShapeDtypeStruct((32, G1 * 16), jnp.float32),
    ],
    scratch_types=[
        pltpu.VMEM((NPT * 16,), jnp.float32),
        pltpu.VMEM((NPT * 16,), jnp.float32),
        pltpu.VMEM((NPT * 16,), jnp.float32),
        pltpu.VMEM((NPT,), jnp.float32),
        pltpu.VMEM((16,), jnp.float32),
        pltpu.VMEM((NPT,), jnp.int32),
        pltpu.VMEM((G1 * 16,), jnp.float32),
        pltpu.VMEM((G1 * 16,), jnp.float32),
        pltpu.VMEM((G1 * 16,), jnp.float32),
    ],
)
def _sc_pool(s3_hbm, g3_hbm, dinv_hbm, b3_hbm, bidx_hbm, maxo, sumo, cnto,
             s0_v, s1_v, g_v, dv_v, b3_v, b_v, macc, sacc, cacc):
    c = lax.axis_index("c")
    s = lax.axis_index("s")
    wid = c * 16 + s
    base = wid * NPT * 16
    pltpu.sync_copy(s3_hbm.at[0, pl.ds(base, NPT * 16)], s0_v)
    pltpu.sync_copy(s3_hbm.at[1, pl.ds(base, NPT * 16)], s1_v)
    pltpu.sync_copy(g3_hbm.at[pl.ds(base, NPT * 16)], g_v)
    pltpu.sync_copy(dinv_hbm.at[pl.ds(wid * NPT, NPT)], dv_v)
    pltpu.sync_copy(b3_hbm, b3_v)
    pltpu.sync_copy(bidx_hbm.at[pl.ds(wid * NPT, NPT)], b_v)
    neg = jnp.full((16,), -jnp.inf, jnp.float32)
    zero = jnp.zeros((16,), jnp.float32)
    one = jnp.ones((16,), jnp.float32)

    def init(j, carry):
        macc[pl.ds(j * 16, 16)] = neg
        sacc[pl.ds(j * 16, 16)] = zero
        cacc[pl.ds(j * 16, 16)] = zero
        return carry

    lax.fori_loop(0, G1, init, 0)
    bvec = b3_v[...]

    def body(i16, carry):
        bs = b_v[pl.ds(i16 * 16, 16)]
        dvs = dv_v[pl.ds(i16 * 16, 16)]
        for l in range(16):
            off = bs[l] * 16
            r = pl.ds((i16 * 16 + l) * 16, 16)
            z = dvs[l] * (s0_v[r] + s1_v[r] + g_v[r]) + bvec
            e = jnp.exp(2.0 * z)
            v = 1.0 - 2.0 / (e + 1.0)
            macc[pl.ds(off, 16)] = jnp.maximum(macc[pl.ds(off, 16)], v)
            sacc[pl.ds(off, 16)] = sacc[pl.ds(off, 16)] + v
            cacc[pl.ds(off, 16)] = cacc[pl.ds(off, 16)] + one
        return carry

    lax.fori_loop(0, NPT // 16, body, 0)
    pltpu.sync_copy(macc, maxo.at[wid])
    pltpu.sync_copy(sacc, sumo.at[wid])
    pltpu.sync_copy(cacc, cnto.at[wid])


def _tc_first(x_p, wt0, deg2):
    def body(x_ref, w_ref, d_ref, g_ref, dinv_ref):
        pid = pl.program_id(0)
        deg = d_ref[0, :, 0:1] + d_ref[1, :, 0:1] + 1.0
        rid = pid * BR + lax.broadcasted_iota(jnp.int32, (BR, 1), 0)
        dinv = jnp.where(rid < N, lax.rsqrt(deg), 0.0)
        g = dinv * jnp.dot(x_ref[...], w_ref[...],
                           preferred_element_type=jnp.float32)
        g_ref[0] = g[:, :64]
        g_ref[1] = g[:, 64:]
        dinv_ref[...] = dinv

    return pl.pallas_call(
        body,
        grid=(NP // BR,),
        in_specs=[
            pl.BlockSpec((BR, 128), lambda i: (i, 0)),
            pl.BlockSpec((128, 128), lambda i: (0, 0)),
            pl.BlockSpec((2, BR, 16), lambda i: (0, i, 0)),
        ],
        out_specs=[
            pl.BlockSpec((2, BR, 64), lambda i: (0, i, 0)),
            pl.BlockSpec((BR, 1), lambda i: (i, 0)),
        ],
        out_shape=[
            jax.ShapeDtypeStruct((2, NP, 64), jnp.float32),
            jax.ShapeDtypeStruct((NP, 1), jnp.float32),
        ],
    )(x_p, wt0, deg2)


def _tc_mid_split(s2, gprev, dinv, bvec, wt, Dn):
    def body(s_ref, g_ref, d_ref, b_ref, w_ref, o_ref):
        dv = d_ref[...]
        sfull = jnp.concatenate([s_ref[0], s_ref[1]], axis=1)
        gfull = jnp.concatenate([g_ref[0], g_ref[1]], axis=1)
        t = jnp.tanh(dv * (sfull + gfull) + b_ref[...])
        o_ref[...] = dv * jnp.dot(t, w_ref[...], preferred_element_type=jnp.float32)

    return pl.pallas_call(
        body,
        grid=(NP // BR,),
        in_specs=[
            pl.BlockSpec((2, BR, 64), lambda i: (0, i, 0)),
            pl.BlockSpec((2, BR, 64), lambda i: (0, i, 0)),
            pl.BlockSpec((BR, 1), lambda i: (i, 0)),
            pl.BlockSpec((1, 128), lambda i: (0, 0)),
            pl.BlockSpec((128, Dn), lambda i: (0, 0)),
        ],
        out_specs=pl.BlockSpec((BR, Dn), lambda i: (i, 0)),
        out_shape=jax.ShapeDtypeStruct((NP, Dn), jnp.float32),
    )(s2, gprev, dinv, bvec, wt)


def _tc_mid(s2, gprev, dinv, bvec, wt, Dp, Dn):
    def body(s_ref, g_ref, d_ref, b_ref, w_ref, o_ref):
        dv = d_ref[...]
        t = jnp.tanh(dv * (s_ref[0] + s_ref[1] + g_ref[...]) + b_ref[...])
        o_ref[...] = dv * jnp.dot(t, w_ref[...], preferred_element_type=jnp.float32)

    return pl.pallas_call(
        body,
        grid=(NP // BR,),
        in_specs=[
            pl.BlockSpec((2, BR, Dp), lambda i: (0, i, 0)),
            pl.BlockSpec((BR, Dp), lambda i: (i, 0)),
            pl.BlockSpec((BR, 1), lambda i: (i, 0)),
            pl.BlockSpec((1, Dp), lambda i: (0, 0)),
            pl.BlockSpec((Dp, Dn), lambda i: (0, 0)),
        ],
        out_specs=pl.BlockSpec((BR, Dn), lambda i: (i, 0)),
        out_shape=jax.ShapeDtypeStruct((NP, Dn), jnp.float32),
    )(s2, gprev, dinv, bvec, wt)


def _tc_final(maxp, sump, cntp, wmax, wmean, bout2):
    def body(m_ref, s_ref, c_ref, wx_ref, wm_ref, b_ref, o_ref):
        mx = jnp.max(m_ref[...], axis=0)[:G, :]
        sm = jnp.sum(s_ref[...], axis=0)[:G, :]
        ct = jnp.sum(c_ref[...], axis=0)[:G, :]
        mean = sm / jnp.maximum(ct, 1.0)
        o_ref[...] = (jnp.dot(mx, wx_ref[...], preferred_element_type=jnp.float32)
                      + jnp.dot(mean, wm_ref[...], preferred_element_type=jnp.float32)
                      + b_ref[...])

    full3 = pl.BlockSpec((32, G1, 16), lambda: (0, 0, 0))
    return pl.pallas_call(
        body,
        in_specs=[
            full3, full3, full3,
            pl.BlockSpec((16, 1), lambda: (0, 0)),
            pl.BlockSpec((16, 1), lambda: (0, 0)),
            pl.BlockSpec((1, 1), lambda: (0, 0)),
        ],
        out_specs=pl.BlockSpec((G, 1), lambda: (0, 0)),
        out_shape=jax.ShapeDtypeStruct((G, 1), jnp.float32),
    )(maxp, sump, cntp, wmax, wmean, bout2)


def kernel(x, edge_index, batch_index, W0, b0, W1, b1, W2, b2, W3, b3, Wout, bout):
    f32 = jnp.float32
    x_p = jnp.pad(x, ((0, NP - N), (0, 0)))
    rows = edge_index[0]
    cols = edge_index[1]
    pad_ids = (N + (jnp.arange(EP - E, dtype=jnp.int32) % (NP - N))).astype(jnp.int32)
    rows_f = jnp.concatenate([rows, pad_ids])
    cols_f = jnp.concatenate([cols, pad_ids])

    def chunked(flat, chd):
        nchd = EPT // chd
        dummy = (N + (jnp.arange(32 * 4 * chd, dtype=jnp.int32) % (NP - N))
                 ).reshape(32, 4, chd).astype(jnp.int32)
        return jnp.concatenate([flat.reshape(32, nchd, chd), dummy], axis=1)

    rows_p = chunked(rows_f, CH)
    cols_p = chunked(cols_f, CH)
    dummy0 = (N + (jnp.arange(16 * 4 * CH, dtype=jnp.int32) % (NP - N))
              ).reshape(16, 4, CH).astype(jnp.int32)
    rows0 = jnp.concatenate([rows_f.reshape(16, NCH0, CH), dummy0], axis=1)
    cols0 = jnp.concatenate([cols_f.reshape(16, NCH0, CH), dummy0], axis=1)
    bidx_p = jnp.concatenate(
        [batch_index.astype(jnp.int32), jnp.full((NP - N,), G, jnp.int32)])
    ones16 = jnp.ones((CH, 16), f32)
    zeros16 = jnp.zeros((NP, 16), f32)

    zeros64 = jnp.zeros((NP, 64), f32)
    deg2 = _sc_degree(cols_p, ones16, zeros16)
    g0, dinv = _tc_first(x_p, W0.T, deg2)
    s0 = _sc_scatter0(g0, rows0, cols0, zeros64)
    g1 = _tc_mid_split(s0, g0, dinv, b0.reshape(1, -1), W1.T, 64)
    s1 = _sc_scatter[64](g1, rows_p, cols_p, zeros64)
    g2 = _tc_mid(s1, g1, dinv, b1.reshape(1, -1), W2.T, 64, 32)
    s2 = _sc_scatter[32](g2, rows_p, cols_p, jnp.zeros((NP, 32), f32))
    g3 = _tc_mid(s2, g2, dinv, b2.reshape(1, -1), W3.T, 32, 16)
    s3 = _sc_scatter[16](g3, rows_p, cols_p, zeros16)

    mx, sm, ct = _sc_pool(s3.reshape(2, NP * 16), g3.reshape(-1),
                          dinv.reshape(-1), b3.astype(f32), bidx_p)
    out = _tc_final(mx.reshape(32, G1, 16), sm.reshape(32, G1, 16),
                    ct.reshape(32, G1, 16),
                    Wout[:, :16].T, Wout[:, 16:].T, bout.reshape(1, 1))
    return out


# speedup vs baseline: 1.1324x; 1.0156x over previous
import functools

import jax
import jax.numpy as jnp
from jax import lax
from jax.experimental import pallas as pl
from jax.experimental.pallas import tpu as pltpu
from jax.experimental.pallas import tpu_sc as plsc

N = 10000
NP = 10240
E = 320000
G = 64
G1 = 72
CH = 128
NCH = 80
NCH2 = NCH + 8
EPT = NCH * CH
EP = 32 * EPT
NPT = NP // 32
RPT = NP // 16
BR = 1024

_mesh = plsc.VectorSubcoreMesh(core_axis_name="c", subcore_axis_name="s")
_sc_params = pltpu.CompilerParams(use_tc_tiling_on_sc=False)


@functools.partial(
    pl.kernel,
    mesh=_mesh,
    compiler_params=_sc_params,
    out_type=jax.ShapeDtypeStruct((2, NP, 16), jnp.float32),
    scratch_types=[
        pltpu.VMEM((NCH2, CH), jnp.int32),
        pltpu.VMEM((CH, 16), jnp.float32),
        pltpu.VMEM_SHARED((NP, 16), jnp.float32),
    ] + [pltpu.SemaphoreType.DMA for _ in range(8)],
)
def _sc_degree(cols_hbm, ones_hbm, zeros_hbm, out_hbm, cols_v, ones_v, acc,
               *sems):
    c = lax.axis_index("c")
    s = lax.axis_index("s")
    wid = c * 16 + s
    pltpu.sync_copy(zeros_hbm.at[pl.ds(s * RPT, RPT)], acc.at[pl.ds(s * RPT, RPT)])
    pltpu.sync_copy(cols_hbm.at[wid], cols_v)
    pltpu.sync_copy(ones_hbm, ones_v)
    plsc.subcore_barrier()

    def body(jj, carry):
        j = 8 * jj
        for b in range(8):
            pltpu.async_copy(ones_v, acc.at[cols_v.at[j + b]], sems[b],
                             add=True)
        for b in range(8):
            pltpu.make_async_copy(ones_hbm, ones_v, sems[b]).wait()
        return carry

    lax.fori_loop(0, NCH // 8, body, 0)
    plsc.subcore_barrier()
    pltpu.sync_copy(acc.at[pl.ds(s * RPT, RPT)], out_hbm.at[c, pl.ds(s * RPT, RPT)])


def _chunk(D):
    return 128


def _nbuf(D):
    return 8 if D <= 32 else 4


def _make_sc_scatter(D):
    CHD = _chunk(D)
    NB = _nbuf(D)
    NCHD = EPT // CHD
    NCHD2 = NCHD + 8

    @functools.partial(
        pl.kernel,
        mesh=_mesh,
        compiler_params=_sc_params,
        out_type=jax.ShapeDtypeStruct((2, NP, D), jnp.float32),
        scratch_types=[
            pltpu.VMEM((NCHD2, CHD), jnp.int32),
            pltpu.VMEM((NCHD2, CHD), jnp.int32),
        ] + [pltpu.VMEM((CHD, D), jnp.float32) for _ in range(NB)] + [
            pltpu.VMEM_SHARED((NP, D), jnp.float32),
        ] + [pltpu.SemaphoreType.DMA for _ in range(2 * NB)],
    )
    def scat(g_hbm, rows_hbm, cols_hbm, zeros_hbm, out_hbm,
             rows_v, cols_v, *rest):
        bufs = rest[:NB]
        acc = rest[NB]
        gsem = rest[NB + 1:2 * NB + 1]
        ssem = rest[2 * NB + 1:]
        c = lax.axis_index("c")
        s = lax.axis_index("s")
        wid = c * 16 + s
        pltpu.sync_copy(zeros_hbm.at[pl.ds(s * RPT, RPT)], acc.at[pl.ds(s * RPT, RPT)])
        pltpu.sync_copy(rows_hbm.at[wid], rows_v)
        pltpu.sync_copy(cols_hbm.at[wid], cols_v)
        plsc.subcore_barrier()

        for b in range(NB):
            pltpu.async_copy(g_hbm.at[rows_v.at[b]], bufs[b], gsem[b])

        def body(jj, carry):
            j = NB * jj
            for b in range(NB):
                pltpu.make_async_copy(g_hbm.at[rows_v.at[j + b]], bufs[b],
                                      gsem[b]).wait()
                pltpu.async_copy(bufs[b], acc.at[cols_v.at[j + b]], ssem[b],
                                 add=True)
            for b in range(NB):
                pltpu.make_async_copy(g_hbm.at[rows_v.at[j + b]], bufs[b],
                                      ssem[b]).wait()
                pltpu.async_copy(g_hbm.at[rows_v.at[j + b + NB]], bufs[b],
                                 gsem[b])
            return carry

        lax.fori_loop(0, NCHD // NB, body, 0)
        for b in range(NB):
            pltpu.make_async_copy(g_hbm.at[rows_v.at[NCHD + b]], bufs[b],
                                  gsem[b]).wait()
        plsc.subcore_barrier()
        pltpu.sync_copy(acc.at[pl.ds(s * RPT, RPT)], out_hbm.at[c, pl.ds(s * RPT, RPT)])

    return scat


_sc_scatter = {D: _make_sc_scatter(D) for D in (64, 32, 16)}

EPT0 = EP // 16
NCH0 = EPT0 // CH
NCH02 = NCH0 + 4
_NB0 = 4


@functools.partial(
    pl.kernel,
    mesh=_mesh,
    compiler_params=_sc_params,
    out_type=jax.ShapeDtypeStruct((2, NP, 64), jnp.float32),
    scratch_types=[
        pltpu.VMEM((NCH02, CH), jnp.int32),
        pltpu.VMEM((NCH02, CH), jnp.int32),
    ] + [pltpu.VMEM((CH, 64), jnp.float32) for _ in range(_NB0)] + [
        pltpu.VMEM_SHARED((NP, 64), jnp.float32),
    ] + [pltpu.SemaphoreType.DMA for _ in range(2 * _NB0)],
)
def _sc_scatter0(g_hbm, rows_hbm, cols_hbm, zeros_hbm, out_hbm,
                 rows_v, cols_v, *rest):
    bufs = rest[:_NB0]
    acc = rest[_NB0]
    gsem = rest[_NB0 + 1:2 * _NB0 + 1]
    ssem = rest[2 * _NB0 + 1:]
    c = lax.axis_index("c")
    s = lax.axis_index("s")
    tab = g_hbm.at[c]
    pltpu.sync_copy(zeros_hbm.at[pl.ds(s * RPT, RPT)], acc.at[pl.ds(s * RPT, RPT)])
    pltpu.sync_copy(rows_hbm.at[s], rows_v)
    pltpu.sync_copy(cols_hbm.at[s], cols_v)
    plsc.subcore_barrier()

    for b in range(_NB0):
        pltpu.async_copy(tab.at[rows_v.at[b]], bufs[b], gsem[b])

    def body(jj, carry):
        j = _NB0 * jj
        for b in range(_NB0):
            pltpu.make_async_copy(tab.at[rows_v.at[j + b]], bufs[b],
                                  gsem[b]).wait()
            pltpu.async_copy(bufs[b], acc.at[cols_v.at[j + b]], ssem[b],
                             add=True)
        for b in range(_NB0):
            pltpu.make_async_copy(tab.at[rows_v.at[j + b]], bufs[b],
                                  ssem[b]).wait()
            pltpu.async_copy(tab.at[rows_v.at[j + b + _NB0]], bufs[b],
                             gsem[b])
        return carry

    lax.fori_loop(0, NCH0 // _NB0, body, 0)
    for b in range(_NB0):
        pltpu.make_async_copy(tab.at[rows_v.at[NCH0 + b]], bufs[b],
                              gsem[b]).wait()
    plsc.subcore_barrier()
    pltpu.sync_copy(acc.at[pl.ds(s * RPT, RPT)], out_hbm.at[c, pl.ds(s * RPT, RPT)])


@functools.partial(
    pl.kernel,
    mesh=_mesh,
    out_type=[
        jax.ShapeDtypeStruct((32, G1 * 16), jnp.float32),
        jax.ShapeDtypeStruct((32, G1 * 16), jnp.float32),
        jax.ShapeDtypeStruct((32, G1 * 16), jnp.float32),
    ],
    scratch_types=[
        pltpu.VMEM((NPT * 16,), jnp.float32),
        pltpu.VMEM((NPT * 16,), jnp.float32),
        pltpu.VMEM((NPT * 16,), jnp.float32),
        pltpu.VMEM((NPT,), jnp.float32),
        pltpu.VMEM((16,), jnp.float32),
        pltpu.VMEM((NPT,), jnp.int32),
        pltpu.VMEM((G1 * 16,), jnp.float32),
        pltpu.VMEM((G1 * 16,), jnp.float32),
        pltpu.VMEM((G1 * 16,), jnp.float32),
    ],
)
def _sc_pool(s3_hbm, g3_hbm, dinv_hbm, b3_hbm, bidx_hbm, maxo, sumo, cnto,
             s0_v, s1_v, g_v, dv_v, b3_v, b_v, macc, sacc, cacc):
    c = lax.axis_index("c")
    s = lax.axis_index("s")
    wid = c * 16 + s
    base = wid * NPT * 16
    pltpu.sync_copy(s3_hbm.at[0, pl.ds(base, NPT * 16)], s0_v)
    pltpu.sync_copy(s3_hbm.at[1, pl.ds(base, NPT * 16)], s1_v)
    pltpu.sync_copy(g3_hbm.at[pl.ds(base, NPT * 16)], g_v)
    pltpu.sync_copy(dinv_hbm.at[pl.ds(wid * NPT, NPT)], dv_v)
    pltpu.sync_copy(b3_hbm, b3_v)
    pltpu.sync_copy(bidx_hbm.at[pl.ds(wid * NPT, NPT)], b_v)
    neg = jnp.full((16,), -jnp.inf, jnp.float32)
    zero = jnp.zeros((16,), jnp.float32)
    one = jnp.ones((16,), jnp.float32)

    def init(j, carry):
        macc[pl.ds(j * 16, 16)] = neg
        sacc[pl.ds(j * 16, 16)] = zero
        cacc[pl.ds(j * 16, 16)] = zero
        return carry

    lax.fori_loop(0, G1, init, 0)
    bvec = b3_v[...]

    def body(i16, carry):
        bs = b_v[pl.ds(i16 * 16, 16)]
        dvs = dv_v[pl.ds(i16 * 16, 16)]
        for l in range(16):
            off = bs[l] * 16
            r = pl.ds((i16 * 16 + l) * 16, 16)
            z = dvs[l] * (s0_v[r] + s1_v[r] + g_v[r]) + bvec
            e = jnp.exp(2.0 * z)
            v = 1.0 - 2.0 / (e + 1.0)
            macc[pl.ds(off, 16)] = jnp.maximum(macc[pl.ds(off, 16)], v)
            sacc[pl.ds(off, 16)] = sacc[pl.ds(off, 16)] + v
            cacc[pl.ds(off, 16)] = cacc[pl.ds(off, 16)] + one
        return carry

    lax.fori_loop(0, NPT // 16, body, 0)
    pltpu.sync_copy(macc, maxo.at[wid])
    pltpu.sync_copy(sacc, sumo.at[wid])
    pltpu.sync_copy(cacc, cnto.at[wid])


def _tc_first(x_p, wt0, deg2):
    def body(x_ref, w_ref, d_ref, g_ref, dinv_ref):
        pid = pl.program_id(0)
        deg = d_ref[0, :, 0:1] + d_ref[1, :, 0:1] + 1.0
        rid = pid * BR + lax.broadcasted_iota(jnp.int32, (BR, 1), 0)
        dinv = jnp.where(rid < N, lax.rsqrt(deg), 0.0)
        g = dinv * jnp.dot(x_ref[...], w_ref[...],
                           preferred_element_type=jnp.float32)
        g_ref[0] = g[:, :64]
        g_ref[1] = g[:, 64:]
        dinv_ref[...] = dinv

    return pl.pallas_call(
        body,
        grid=(NP // BR,),
        in_specs=[
            pl.BlockSpec((BR, 128), lambda i: (i, 0)),
            pl.BlockSpec((128, 128), lambda i: (0, 0)),
            pl.BlockSpec((2, BR, 16), lambda i: (0, i, 0)),
        ],
        out_specs=[
            pl.BlockSpec((2, BR, 64), lambda i: (0, i, 0)),
            pl.BlockSpec((BR, 1), lambda i: (i, 0)),
        ],
        out_shape=[
            jax.ShapeDtypeStruct((2, NP, 64), jnp.float32),
            jax.ShapeDtypeStruct((NP, 1), jnp.float32),
        ],
    )(x_p, wt0, deg2)


def _tc_mid_split(s2, gprev, dinv, bvec, wt, Dn):
    def body(s_ref, g_ref, d_ref, b_ref, w_ref, o_ref):
        dv = d_ref[...]
        sfull = jnp.concatenate([s_ref[0], s_ref[1]], axis=1)
        gfull = jnp.concatenate([g_ref[0], g_ref[1]], axis=1)
        t = jnp.tanh(dv * (sfull + gfull) + b_ref[...])
        o_ref[...] = dv * jnp.dot(t, w_ref[...], preferred_element_type=jnp.float32)

    return pl.pallas_call(
        body,
        grid=(NP // BR,),
        in_specs=[
            pl.BlockSpec((2, BR, 64), lambda i: (0, i, 0)),
            pl.BlockSpec((2, BR, 64), lambda i: (0, i, 0)),
            pl.BlockSpec((BR, 1), lambda i: (i, 0)),
            pl.BlockSpec((1, 128), lambda i: (0, 0)),
            pl.BlockSpec((128, Dn), lambda i: (0, 0)),
        ],
        out_specs=pl.BlockSpec((BR, Dn), lambda i: (i, 0)),
        out_shape=jax.ShapeDtypeStruct((NP, Dn), jnp.float32),
    )(s2, gprev, dinv, bvec, wt)


def _tc_mid(s2, gprev, dinv, bvec, wt, Dp, Dn):
    def body(s_ref, g_ref, d_ref, b_ref, w_ref, o_ref):
        dv = d_ref[...]
        t = jnp.tanh(dv * (s_ref[0] + s_ref[1] + g_ref[...]) + b_ref[...])
        o_ref[...] = dv * jnp.dot(t, w_ref[...], preferred_element_type=jnp.float32)

    return pl.pallas_call(
        body,
        grid=(NP // BR,),
        in_specs=[
            pl.BlockSpec((2, BR, Dp), lambda i: (0, i, 0)),
            pl.BlockSpec((BR, Dp), lambda i: (i, 0)),
            pl.BlockSpec((BR, 1), lambda i: (i, 0)),
            pl.BlockSpec((1, Dp), lambda i: (0, 0)),
            pl.BlockSpec((Dp, Dn), lambda i: (0, 0)),
        ],
        out_specs=pl.BlockSpec((BR, Dn), lambda i: (i, 0)),
        out_shape=jax.ShapeDtypeStruct((NP, Dn), jnp.float32),
    )(s2, gprev, dinv, bvec, wt)


def _tc_final(maxp, sump, cntp, wmax, wmean, bout2):
    def body(m_ref, s_ref, c_ref, wx_ref, wm_ref, b_ref, o_ref):
        mx = jnp.max(m_ref[...], axis=0)[:G, :]
        sm = jnp.sum(s_ref[...], axis=0)[:G, :]
        ct = jnp.sum(c_ref[...], axis=0)[:G, :]
        mean = sm / jnp.maximum(ct, 1.0)
        o_ref[...] = (jnp.dot(mx, wx_ref[...], preferred_element_type=jnp.float32)
                      + jnp.dot(mean, wm_ref[...], preferred_element_type=jnp.float32)
                      + b_ref[...])

    full3 = pl.BlockSpec((32, G1, 16), lambda: (0, 0, 0))
    return pl.pallas_call(
        body,
        in_specs=[
            full3, full3, full3,
            pl.BlockSpec((16, 1), lambda: (0, 0)),
            pl.BlockSpec((16, 1), lambda: (0, 0)),
            pl.BlockSpec((1, 1), lambda: (0, 0)),
        ],
        out_specs=pl.BlockSpec((G, 1), lambda: (0, 0)),
        out_shape=jax.ShapeDtypeStruct((G, 1), jnp.float32),
    )(maxp, sump, cntp, wmax, wmean, bout2)


def kernel(x, edge_index, batch_index, W0, b0, W1, b1, W2, b2, W3, b3, Wout, bout):
    f32 = jnp.float32
    x_p = jnp.pad(x, ((0, NP - N), (0, 0)))
    rows = edge_index[0]
    cols = edge_index[1]
    pad_ids = (N + (jnp.arange(EP - E, dtype=jnp.int32) % (NP - N))).astype(jnp.int32)
    rows_f = jnp.concatenate([rows, pad_ids])
    cols_f = jnp.concatenate([cols, pad_ids])

    def chunked(flat, chd):
        nchd = EPT // chd
        dummy = (N + (jnp.arange(32 * 8 * chd, dtype=jnp.int32) % (NP - N))
                 ).reshape(32, 8, chd).astype(jnp.int32)
        return jnp.concatenate([flat.reshape(32, nchd, chd), dummy], axis=1)

    rows_p = chunked(rows_f, CH)
    cols_p = chunked(cols_f, CH)
    dummy0 = (N + (jnp.arange(16 * 4 * CH, dtype=jnp.int32) % (NP - N))
              ).reshape(16, 4, CH).astype(jnp.int32)
    rows0 = jnp.concatenate([rows_f.reshape(16, NCH0, CH), dummy0], axis=1)
    cols0 = jnp.concatenate([cols_f.reshape(16, NCH0, CH), dummy0], axis=1)
    bidx_p = jnp.concatenate(
        [batch_index.astype(jnp.int32), jnp.full((NP - N,), G, jnp.int32)])
    ones16 = jnp.ones((CH, 16), f32)
    zeros16 = jnp.zeros((NP, 16), f32)

    zeros64 = jnp.zeros((NP, 64), f32)
    deg2 = _sc_degree(cols_p, ones16, zeros16)
    g0, dinv = _tc_first(x_p, W0.T, deg2)
    s0 = _sc_scatter0(g0, rows0, cols0, zeros64)
    g1 = _tc_mid_split(s0, g0, dinv, b0.reshape(1, -1), W1.T, 64)
    s1 = _sc_scatter[64](g1, rows_p, cols_p, zeros64)
    g2 = _tc_mid(s1, g1, dinv, b1.reshape(1, -1), W2.T, 64, 32)
    s2 = _sc_scatter[32](g2, rows_p, cols_p, jnp.zeros((NP, 32), f32))
    g3 = _tc_mid(s2, g2, dinv, b2.reshape(1, -1), W3.T, 32, 16)
    s3 = _sc_scatter[16](g3, rows_p, cols_p, zeros16)

    mx, sm, ct = _sc_pool(s3.reshape(2, NP * 16), g3.reshape(-1),
                          dinv.reshape(-1), b3.astype(f32), bidx_p)
    out = _tc_final(mx.reshape(32, G1, 16), sm.reshape(32, G1, 16),
                    ct.reshape(32, G1, 16),
                    Wout[:, :16].T, Wout[:, 16:].T, bout.reshape(1, 1))
    return out
